# baseline (speedup 1.0000x reference)
import functools

import jax
import jax.numpy as jnp
from jax import lax
from jax.experimental import pallas as pl
from jax.experimental.pallas import tpu as pltpu
from jax.experimental.pallas import tpu_sc as plsc


def _table_body(e_ref, wt_ref, b_ref, o_ref):
    o_ref[...] = (
        jnp.dot(e_ref[...], wt_ref[...], preferred_element_type=jnp.float32)
        + b_ref[...]
    )


def _build_table(embed, w_t, b_row):
    v, _ = embed.shape
    d = w_t.shape[1]
    return pl.pallas_call(
        _table_body,
        out_shape=jax.ShapeDtypeStruct((v, d), jnp.float32),
    )(embed, w_t, b_row)


@functools.cache
def _make_gather(n_rows, d):
    info = plsc.get_sparse_core_info()
    nc, ns = info.num_cores, info.num_subcores
    nw = nc * ns
    assert n_rows % nw == 0
    rows_per_w = n_rows // nw
    chunk = 80
    assert rows_per_w % chunk == 0 and chunk % 8 == 0 and chunk <= 128
    n_chunks = rows_per_w // chunk
    mesh = plsc.VectorSubcoreMesh(core_axis_name="c", subcore_axis_name="s")

    @functools.partial(
        pl.kernel,
        mesh=mesh,
        compiler_params=pltpu.CompilerParams(use_tc_tiling_on_sc=False),
        out_type=jax.ShapeDtypeStruct((n_rows, d), jnp.float32),
        scratch_types=[
            pltpu.VMEM((rows_per_w,), jnp.int32),
            pltpu.VMEM((chunk, d), jnp.float32),
            pltpu.SemaphoreType.DMA,
        ],
    )
    def gather(table_hbm, idx_hbm, out_hbm, idx_v, rows_v, sem):
        wid = lax.axis_index("s") * nc + lax.axis_index("c")
        base = wid * rows_per_w
        pltpu.sync_copy(idx_hbm.at[pl.ds(base, rows_per_w)], idx_v)

        def body(c, carry):
            idx_sl = idx_v.at[pl.ds(c * chunk, chunk)]
            pltpu.async_copy(table_hbm.at[idx_sl], rows_v, sem).wait()
            pltpu.sync_copy(rows_v, out_hbm.at[pl.ds(base + c * chunk, chunk)])
            return carry

        lax.fori_loop(0, n_chunks, body, 0)

    return gather


def kernel(input_ids, embed, W, b):
    bsz, seq = input_ids.shape
    vocab = W.shape[0]
    ids = input_ids.reshape(-1).astype(jnp.int32)
    table = _build_table(embed, W.T, b.reshape(1, vocab))
    out = _make_gather(bsz * seq, vocab)(table, ids)
    return out.reshape(bsz, seq, vocab)


# speedup vs baseline: 1.0129x; 1.0129x over previous
import functools

import jax
import jax.numpy as jnp
from jax import lax
from jax.experimental import pallas as pl
from jax.experimental.pallas import tpu as pltpu
from jax.experimental.pallas import tpu_sc as plsc


def _table_body(e_ref, wt_ref, b_ref, o_ref):
    o_ref[...] = (
        jnp.dot(e_ref[...], wt_ref[...], preferred_element_type=jnp.float32)
        + b_ref[...]
    )


def _build_table(embed, w_t, b_row):
    v, _ = embed.shape
    d = w_t.shape[1]
    return pl.pallas_call(
        _table_body,
        out_shape=jax.ShapeDtypeStruct((v, d), jnp.float32),
    )(embed, w_t, b_row)


@functools.cache
def _make_gather(n_rows, d):
    info = plsc.get_sparse_core_info()
    nc, ns = info.num_cores, info.num_subcores
    nw = nc * ns
    assert n_rows % nw == 0
    rows_per_w = n_rows // nw
    chunk = 40
    assert rows_per_w % chunk == 0 and chunk % 8 == 0 and chunk <= 128
    n_chunks = rows_per_w // chunk
    assert n_chunks % 2 == 0
    mesh = plsc.VectorSubcoreMesh(core_axis_name="c", subcore_axis_name="s")

    @functools.partial(
        pl.kernel,
        mesh=mesh,
        compiler_params=pltpu.CompilerParams(use_tc_tiling_on_sc=False),
        out_type=jax.ShapeDtypeStruct((n_rows, d), jnp.float32),
        scratch_types=[
            pltpu.VMEM((rows_per_w,), jnp.int32),
            pltpu.VMEM((chunk, d), jnp.float32),
            pltpu.VMEM((chunk, d), jnp.float32),
            pltpu.SemaphoreType.DMA,
            pltpu.SemaphoreType.DMA,
        ],
    )
    def gather(table_hbm, idx_hbm, out_hbm, idx_v, rows0, rows1, sem0, sem1):
        wid = lax.axis_index("s") * nc + lax.axis_index("c")
        base = wid * rows_per_w
        pltpu.sync_copy(idx_hbm.at[pl.ds(base, rows_per_w)], idx_v)

        def fire(c, buf, sem):
            idx_sl = idx_v.at[pl.ds(c * chunk, chunk)]
            pltpu.async_copy(table_hbm.at[idx_sl], buf, sem)

        def drain(c, buf, sem):
            idx_sl = idx_v.at[pl.ds(c * chunk, chunk)]
            pltpu.make_async_copy(table_hbm.at[idx_sl], buf, sem).wait()

        fire(0, rows0, sem0)
        fire(1, rows1, sem1)

        def body(i, carry):
            c0 = 2 * i
            for off, (buf, sem) in enumerate(((rows0, sem0), (rows1, sem1))):
                c = c0 + off
                drain(c, buf, sem)
                pltpu.sync_copy(buf, out_hbm.at[pl.ds(base + c * chunk, chunk)])

                @pl.when(c + 2 < n_chunks)
                def _():
                    fire(c + 2, buf, sem)

            return carry

        lax.fori_loop(0, n_chunks // 2, body, 0)

    return gather


def kernel(input_ids, embed, W, b):
    bsz, seq = input_ids.shape
    vocab = W.shape[0]
    ids = input_ids.reshape(-1).astype(jnp.int32)
    table = _build_table(embed, W.T, b.reshape(1, vocab))
    out = _make_gather(bsz * seq, vocab)(table, ids)
    return out.reshape(bsz, seq, vocab)


# speedup vs baseline: 1.1065x; 1.0924x over previous
import functools

import jax
import jax.numpy as jnp
from jax import lax
from jax.experimental import pallas as pl
from jax.experimental.pallas import tpu as pltpu
from jax.experimental.pallas import tpu_sc as plsc


def _table_body(e_ref, wt_ref, b_ref, o_ref):
    o_ref[...] = (
        jnp.dot(e_ref[...], wt_ref[...], preferred_element_type=jnp.float32)
        + b_ref[...]
    )


def _build_table(embed, w_t, b_row):
    v, _ = embed.shape
    d = w_t.shape[1]
    return pl.pallas_call(
        _table_body,
        out_shape=jax.ShapeDtypeStruct((v, d), jnp.float32),
    )(embed, w_t, b_row)


@functools.cache
def _make_gather(bsz, seq, vocab):
    info = plsc.get_sparse_core_info()
    nc, ns = info.num_cores, info.num_subcores
    nw = nc * ns
    assert bsz % nw == 0
    bat_per_w = bsz // nw
    tok_per_w = bat_per_w * seq
    seq_pad = (seq + 15) // 16 * 16
    n_vecs = seq_pad // 16
    seq_gather = (seq + 7) // 8 * 8
    mesh = plsc.VectorSubcoreMesh(core_axis_name="c", subcore_axis_name="s")

    @functools.partial(
        pl.kernel,
        mesh=mesh,
        compiler_params=pltpu.CompilerParams(use_tc_tiling_on_sc=False),
        out_type=jax.ShapeDtypeStruct((bsz, seq, vocab), jnp.float32),
        scratch_types=[
            pltpu.VMEM((bat_per_w, seq_pad), jnp.int32),
            pltpu.VMEM((24, vocab), jnp.float32),
            pltpu.VMEM((32, vocab), jnp.float32),
            pltpu.VMEM_SHARED((vocab, vocab), jnp.float32),
            pltpu.SemaphoreType.DMA,
            pltpu.SemaphoreType.DMA,
        ],
    )
    def gather(table_hbm, idx_hbm, out_hbm, idx2d, buf_a, buf_b,
               table_sh, sem_a, sem_b):
        cid = lax.axis_index("c")
        sid = lax.axis_index("s")
        wid = sid * nc + cid
        base_b = wid * bat_per_w

        @pl.when(sid == 0)
        def _():
            pltpu.sync_copy(table_hbm, table_sh)

        pltpu.sync_copy(idx_hbm.at[pl.ds(base_b, bat_per_w)], idx2d)
        plsc.subcore_barrier()

        h1 = seq - 24

        def fire_a(c):
            pltpu.async_copy(table_sh.at[idx2d.at[c, pl.ds(0, 24)]], buf_a, sem_a)

        def fire_b(c):
            pltpu.async_copy(table_sh.at[idx2d.at[c, pl.ds(24, 32)]], buf_b, sem_b)

        def drain_a(c):
            pltpu.make_async_copy(
                table_sh.at[idx2d.at[c, pl.ds(0, 24)]], buf_a, sem_a).wait()

        def drain_b(c):
            pltpu.make_async_copy(
                table_sh.at[idx2d.at[c, pl.ds(24, 32)]], buf_b, sem_b).wait()

        fire_a(0)
        fire_b(0)

        def body(c, carry):
            drain_a(c)
            pltpu.sync_copy(buf_a, out_hbm.at[base_b + c, pl.ds(0, 24)])

            @pl.when(c + 1 < bat_per_w)
            def _():
                fire_a(c + 1)

            drain_b(c)
            pltpu.sync_copy(buf_b.at[pl.ds(0, h1)],
                            out_hbm.at[base_b + c, pl.ds(24, h1)])

            @pl.when(c + 1 < bat_per_w)
            def _():
                fire_b(c + 1)

            return carry

        lax.fori_loop(0, bat_per_w, body, 0)

    return gather


def kernel(input_ids, embed, W, b):
    bsz, seq = input_ids.shape
    vocab = W.shape[0]
    seq_pad = (seq + 15) // 16 * 16
    ids = jnp.pad(input_ids.astype(jnp.int32), ((0, 0), (0, seq_pad - seq)))
    table = _build_table(embed, W.T, b.reshape(1, vocab))
    return _make_gather(bsz, seq, vocab)(table, ids)
